# baseline (speedup 1.0000x reference)
import functools

import jax
import jax.numpy as jnp
from jax import lax
from jax.experimental import pallas as pl
from jax.experimental.pallas import tpu as pltpu
from jax.experimental.pallas import tpu_sc as plsc

C0 = 2000
C1 = 20000
HEAD = 2002
HEAD_PAD = 2048
V0 = 18000
V1 = 80000
D1 = 256
D2 = 64
CHUNK = 4096
RB = 512

NC = 2
NS = 16
L = 16


def _gather_rows_body(t_hbm, w20_hbm, w21_hbm, out0_hbm, out1_hbm,
                      t_v, i0_v, i1_v, r0_v, r1_v, sem, *, bpw):
    wid = lax.axis_index("s") * NC + lax.axis_index("c")
    base = wid * bpw
    pltpu.sync_copy(t_hbm.at[pl.ds(base, bpw)], t_v)
    for j in range(bpw // L):
        t = t_v[pl.ds(j * L, L)]
        in0 = (t >= C0) & (t < C1)
        i0_v[pl.ds(j * L, L)] = jnp.where(in0, t - C0,
                                          jnp.bitwise_and(t, 8191))
        i1_v[pl.ds(j * L, L)] = jnp.where(t >= C1, t - C1, t) >> 1
    copies = []
    for k in range(bpw // 128):
        sl = pl.ds(k * 128, 128)
        copies.append(pltpu.async_copy(w20_hbm.at[i0_v.at[sl]],
                                       r0_v.at[sl], sem))
        copies.append(pltpu.async_copy(w21_hbm.at[i1_v.at[sl]],
                                       r1_v.at[sl], sem))
    for c in copies:
        c.wait()
    pltpu.sync_copy(r0_v, out0_hbm.at[pl.ds(base, bpw)])
    pltpu.sync_copy(r1_v, out1_hbm.at[pl.ds(base, bpw)])


def _gather_rows(t32, w20, w21):
    n = t32.shape[0]
    bpw = n // (NC * NS)
    body = functools.partial(_gather_rows_body, bpw=bpw)
    return pl.kernel(
        body,
        out_type=[jax.ShapeDtypeStruct((n, w20.shape[1]), jnp.float32),
                  jax.ShapeDtypeStruct((n, w21.shape[1]), jnp.float32)],
        mesh=plsc.VectorSubcoreMesh(core_axis_name="c", subcore_axis_name="s",
                                    num_cores=NC, num_subcores=NS),
        scratch_types=[pltpu.VMEM((bpw,), jnp.int32),
                       pltpu.VMEM((bpw,), jnp.int32),
                       pltpu.VMEM((bpw,), jnp.int32),
                       pltpu.VMEM((bpw, w20.shape[1]), jnp.float32),
                       pltpu.VMEM((bpw, w21.shape[1]), jnp.float32),
                       pltpu.SemaphoreType.DMA],
    )(t32, w20, w21)


def _main_kernel(x_ref, t_ref, wh_ref, w10_ref, w20_ref, w11_ref, w21_ref,
                 head_ref, lse0_ref, lse1_ref, h0_ref, h1_ref, *, rb):
    x = x_ref[...]
    t = t_ref[0]

    h0 = jax.lax.dot_general(x, w10_ref[...], (((1,), (1,)), ((), ())),
                             preferred_element_type=jnp.float32)
    h0 = h0.astype(jnp.bfloat16)
    h1 = jax.lax.dot_general(x, w11_ref[...], (((1,), (1,)), ((), ())),
                             preferred_element_type=jnp.float32)
    h1 = h1.astype(jnp.bfloat16)
    h0_ref[...] = h0
    h1_ref[...] = h1

    lg = jax.lax.dot_general(x, wh_ref[...], (((1,), (1,)), ((), ())),
                             preferred_element_type=jnp.float32)
    col = jax.lax.broadcasted_iota(jnp.int32, (rb, HEAD_PAD), 1)
    s = jnp.sum(jnp.exp(lg), axis=1, keepdims=True)
    s = s - (HEAD_PAD - HEAD)
    lse_h = jnp.log(s)
    gidx = jnp.where(t < C0, t, jnp.where(t < C1, C0, C0 + 1))
    at_h = jnp.sum(jnp.where(col == gidx, lg, 0.0), axis=1, keepdims=True)
    head_ref[...] = at_h - lse_h

    def tail_lse(h, w_ref, vocab):
        nchunks = w_ref.shape[0] // CHUNK

        def body(c, s):
            w = w_ref[pl.ds(c * CHUNK, CHUNK), :]
            lg = jax.lax.dot_general(h, w, (((1,), (1,)), ((), ())),
                                     preferred_element_type=jnp.float32)
            e = jnp.exp(lg.astype(jnp.bfloat16))
            return s + jnp.sum(e.astype(jnp.float32), axis=1, keepdims=True)

        zero = jnp.zeros((rb, 1), dtype=jnp.float32)
        s = jax.lax.fori_loop(0, nchunks, body, zero)
        s = s - (nchunks * CHUNK - vocab)
        return jnp.log(s)

    lse0_ref[...] = tail_lse(h0, w20_ref, V0)
    lse1_ref[...] = tail_lse(h1, w21_ref, V1)


def _combine_kernel(t_ref, head_ref, lse0_ref, lse1_ref, h0_ref, h1_ref,
                    w0r_ref, w1r_ref, out_ref):
    t = t_ref[0]

    def dot_rows(h, wr):
        wb = wr.astype(jnp.bfloat16).astype(jnp.float32)
        return jnp.sum(h.astype(jnp.float32) * wb, axis=1, keepdims=True)

    w1r = w1r_ref[...]
    odd = jnp.bitwise_and(t, 1) == 1
    w1sel = jnp.where(odd, w1r[:, D2:], w1r[:, :D2])
    local0 = dot_rows(h0_ref[...], w0r_ref[...]) - lse0_ref[...]
    local1 = dot_rows(h1_ref[...], w1sel) - lse1_ref[...]
    in0 = (t >= C0) & (t < C1)
    in1 = t >= C1
    out_ref[...] = head_ref[...] \
        + jnp.where(in0, local0, 0.0) \
        + jnp.where(in1, local1, 0.0)


def _pad_rows(w, n):
    return jnp.pad(w, ((0, n - w.shape[0]), (0, 0)))


@jax.jit
def kernel(input, target, W_head, W1_0, W2_0, W1_1, W2_1):
    n, d = input.shape
    rb = RB
    grid = n // rb

    t32 = target.astype(jnp.int32)
    w21v = W2_1.reshape(V1 // 2, 2 * D2)
    w0rows, w1rows = _gather_rows(t32, W2_0, w21v)

    x = input.astype(jnp.bfloat16)
    t3 = t32.reshape(grid, rb, 1)
    wh = _pad_rows(W_head, HEAD_PAD).astype(jnp.bfloat16)
    w10 = W1_0.astype(jnp.bfloat16)
    w20 = _pad_rows(W2_0, 5 * CHUNK).astype(jnp.bfloat16)
    w11 = W1_1.astype(jnp.bfloat16)
    w21 = _pad_rows(W2_1, 20 * CHUNK).astype(jnp.bfloat16)

    const = lambda shape: pl.BlockSpec(shape, lambda i: (0,) * len(shape))
    row = lambda width, dt: (pl.BlockSpec((rb, width), lambda i: (i, 0)),
                             jax.ShapeDtypeStruct((n, width), dt))
    out_specs, out_shapes = zip(
        row(1, jnp.float32),
        row(1, jnp.float32),
        row(1, jnp.float32),
        row(D1, jnp.bfloat16),
        row(D2, jnp.bfloat16),
    )
    head_t, lse0, lse1, h0, h1 = pl.pallas_call(
        functools.partial(_main_kernel, rb=rb),
        grid=(grid,),
        in_specs=[
            pl.BlockSpec((rb, d), lambda i: (i, 0)),
            pl.BlockSpec((1, rb, 1), lambda i: (i, 0, 0)),
            const(wh.shape),
            const(w10.shape),
            const(w20.shape),
            const(w11.shape),
            const(w21.shape),
        ],
        out_specs=list(out_specs),
        out_shape=list(out_shapes),
        compiler_params=pltpu.CompilerParams(
            dimension_semantics=("arbitrary",),
        ),
    )(x, t3, wh, w10, w20, w11, w21)

    cb = 2048
    cgrid = n // cb
    t3c = t32.reshape(cgrid, cb, 1)
    crow = lambda width: pl.BlockSpec((cb, width), lambda i: (i, 0))
    out = pl.pallas_call(
        _combine_kernel,
        grid=(cgrid,),
        in_specs=[
            pl.BlockSpec((1, cb, 1), lambda i: (i, 0, 0)),
            crow(1), crow(1), crow(1),
            crow(D1), crow(D2), crow(D1), crow(2 * D2),
        ],
        out_specs=crow(1),
        out_shape=jax.ShapeDtypeStruct((n, 1), jnp.float32),
        compiler_params=pltpu.CompilerParams(
            dimension_semantics=("arbitrary",),
        ),
    )(t3c, head_t, lse0, lse1, h0, h1, w0rows, w1rows)

    output = out.reshape(n)
    loss = (-output).mean()
    return output, loss


# speedup vs baseline: 1.0308x; 1.0308x over previous
import functools

import jax
import jax.numpy as jnp
from jax import lax
from jax.experimental import pallas as pl
from jax.experimental.pallas import tpu as pltpu
from jax.experimental.pallas import tpu_sc as plsc

C0 = 2000
C1 = 20000
HEAD = 2002
HEAD_PAD = 2048
V0 = 18000
V1 = 80000
D1 = 256
D2 = 64
CHUNK = 4096
RB = 512

NC = 2
NS = 16
L = 16


def _gather_rows_body(t_hbm, w20_hbm, w21_hbm, out0_hbm, out1_hbm,
                      t_v, i0_v, i1_v, r0_v, r1_v, sem, *, bpw):
    wid = lax.axis_index("s") * NC + lax.axis_index("c")
    base = wid * bpw
    pltpu.sync_copy(t_hbm.at[pl.ds(base, bpw)], t_v)
    for j in range(bpw // L):
        t = t_v[pl.ds(j * L, L)]
        in0 = (t >= C0) & (t < C1)
        i0_v[pl.ds(j * L, L)] = jnp.where(in0, t - C0,
                                          jnp.bitwise_and(t, 8191))
        i1_v[pl.ds(j * L, L)] = jnp.where(t >= C1, t - C1, t) >> 1
    copies = []
    for k in range(bpw // 128):
        sl = pl.ds(k * 128, 128)
        copies.append(pltpu.async_copy(w20_hbm.at[i0_v.at[sl]],
                                       r0_v.at[sl], sem))
        copies.append(pltpu.async_copy(w21_hbm.at[i1_v.at[sl]],
                                       r1_v.at[sl], sem))
    for c in copies:
        c.wait()
    pltpu.sync_copy(r0_v, out0_hbm.at[pl.ds(base, bpw)])
    pltpu.sync_copy(r1_v, out1_hbm.at[pl.ds(base, bpw)])


def _gather_rows(t32, w20, w21):
    n = t32.shape[0]
    bpw = n // (NC * NS)
    body = functools.partial(_gather_rows_body, bpw=bpw)
    return pl.kernel(
        body,
        out_type=[jax.ShapeDtypeStruct((n, w20.shape[1]), jnp.float32),
                  jax.ShapeDtypeStruct((n, w21.shape[1]), jnp.float32)],
        mesh=plsc.VectorSubcoreMesh(core_axis_name="c", subcore_axis_name="s",
                                    num_cores=NC, num_subcores=NS),
        scratch_types=[pltpu.VMEM((bpw,), jnp.int32),
                       pltpu.VMEM((bpw,), jnp.int32),
                       pltpu.VMEM((bpw,), jnp.int32),
                       pltpu.VMEM((bpw, w20.shape[1]), jnp.float32),
                       pltpu.VMEM((bpw, w21.shape[1]), jnp.float32),
                       pltpu.SemaphoreType.DMA],
    )(t32, w20, w21)


def _main_kernel(x_ref, t_ref, wh_ref, w10_ref, w20_ref, w11_ref, w21_ref,
                 head_ref, lse0_ref, lse1_ref, h0_ref, h1_ref, *, rb):
    x = x_ref[...].astype(jnp.bfloat16)
    t = t_ref[0]

    h0 = jax.lax.dot_general(x, w10_ref[...], (((1,), (1,)), ((), ())),
                             preferred_element_type=jnp.float32)
    h0 = h0.astype(jnp.bfloat16)
    h1 = jax.lax.dot_general(x, w11_ref[...], (((1,), (1,)), ((), ())),
                             preferred_element_type=jnp.float32)
    h1 = h1.astype(jnp.bfloat16)
    h0_ref[...] = h0
    h1_ref[...] = h1

    lg = jax.lax.dot_general(x, wh_ref[...], (((1,), (1,)), ((), ())),
                             preferred_element_type=jnp.float32)
    col = jax.lax.broadcasted_iota(jnp.int32, (rb, HEAD_PAD), 1)
    s = jnp.sum(jnp.exp(lg), axis=1, keepdims=True)
    s = s - (HEAD_PAD - HEAD)
    lse_h = jnp.log(s)
    gidx = jnp.where(t < C0, t, jnp.where(t < C1, C0, C0 + 1))
    at_h = jnp.sum(jnp.where(col == gidx, lg, 0.0), axis=1, keepdims=True)
    head_ref[...] = at_h - lse_h

    def tail_lse(h, w_ref, vocab):
        nchunks = w_ref.shape[0] // CHUNK

        def body(c, s):
            w = w_ref[pl.ds(c * CHUNK, CHUNK), :]
            lg = jax.lax.dot_general(h, w, (((1,), (1,)), ((), ())),
                                     preferred_element_type=jnp.float32)
            return s + jnp.sum(jnp.exp(lg), axis=1, keepdims=True)

        zero = jnp.zeros((rb, 1), dtype=jnp.float32)
        s = jax.lax.fori_loop(0, nchunks, body, zero)
        s = s - (nchunks * CHUNK - vocab)
        return jnp.log(s)

    lse0_ref[...] = tail_lse(h0, w20_ref, V0)
    lse1_ref[...] = tail_lse(h1, w21_ref, V1)


def _combine_kernel(t_ref, head_ref, lse0_ref, lse1_ref, h0_ref, h1_ref,
                    w0r_ref, w1r_ref, out_ref):
    t = t_ref[0]

    def dot_rows(h, wr):
        wb = wr.astype(jnp.bfloat16).astype(jnp.float32)
        return jnp.sum(h.astype(jnp.float32) * wb, axis=1, keepdims=True)

    w1r = w1r_ref[...]
    odd = jnp.bitwise_and(t, 1) == 1
    w1sel = jnp.where(odd, w1r[:, D2:], w1r[:, :D2])
    local0 = dot_rows(h0_ref[...], w0r_ref[...]) - lse0_ref[...]
    local1 = dot_rows(h1_ref[...], w1sel) - lse1_ref[...]
    in0 = (t >= C0) & (t < C1)
    in1 = t >= C1
    out_ref[...] = head_ref[...] \
        + jnp.where(in0, local0, 0.0) \
        + jnp.where(in1, local1, 0.0)


def _pad_rows(w, n):
    return jnp.pad(w, ((0, n - w.shape[0]), (0, 0)))


@jax.jit
def kernel(input, target, W_head, W1_0, W2_0, W1_1, W2_1):
    n, d = input.shape
    rb = RB
    grid = n // rb

    t32 = target.astype(jnp.int32)
    x = input
    t3 = t32.reshape(grid, rb, 1)
    wh = _pad_rows(W_head, HEAD_PAD).astype(jnp.bfloat16)
    w10 = W1_0.astype(jnp.bfloat16)
    w20 = _pad_rows(W2_0, 5 * CHUNK).astype(jnp.bfloat16)
    w11 = W1_1.astype(jnp.bfloat16)
    w21 = _pad_rows(W2_1, 20 * CHUNK).astype(jnp.bfloat16)

    const = lambda shape: pl.BlockSpec(shape, lambda i: (0,) * len(shape))
    row = lambda width, dt: (pl.BlockSpec((rb, width), lambda i: (i, 0)),
                             jax.ShapeDtypeStruct((n, width), dt))
    out_specs, out_shapes = zip(
        row(1, jnp.float32),
        row(1, jnp.float32),
        row(1, jnp.float32),
        row(D1, jnp.bfloat16),
        row(D2, jnp.bfloat16),
    )
    head_t, lse0, lse1, h0, h1 = pl.pallas_call(
        functools.partial(_main_kernel, rb=rb),
        grid=(grid,),
        in_specs=[
            pl.BlockSpec((rb, d), lambda i: (i, 0)),
            pl.BlockSpec((1, rb, 1), lambda i: (i, 0, 0)),
            const(wh.shape),
            const(w10.shape),
            const(w20.shape),
            const(w11.shape),
            const(w21.shape),
        ],
        out_specs=list(out_specs),
        out_shape=list(out_shapes),
        compiler_params=pltpu.CompilerParams(
            dimension_semantics=("arbitrary",),
        ),
    )(x, t3, wh, w10, w20, w11, w21)

    w21v = W2_1.reshape(V1 // 2, 2 * D2)
    w0rows, w1rows = _gather_rows(t32, W2_0, w21v)

    cb = 2048
    cgrid = n // cb
    t3c = t32.reshape(cgrid, cb, 1)
    crow = lambda width: pl.BlockSpec((cb, width), lambda i: (i, 0))
    out = pl.pallas_call(
        _combine_kernel,
        grid=(cgrid,),
        in_specs=[
            pl.BlockSpec((1, cb, 1), lambda i: (i, 0, 0)),
            crow(1), crow(1), crow(1),
            crow(D1), crow(D2), crow(D1), crow(2 * D2),
        ],
        out_specs=crow(1),
        out_shape=jax.ShapeDtypeStruct((n, 1), jnp.float32),
        compiler_params=pltpu.CompilerParams(
            dimension_semantics=("arbitrary",),
        ),
    )(t3c, head_t, lse0, lse1, h0, h1, w0rows, w1rows)

    output = out.reshape(n)
    loss = (-output).mean()
    return output, loss


# speedup vs baseline: 1.1197x; 1.0862x over previous
import functools

import jax
import jax.numpy as jnp
from jax import lax
from jax.experimental import pallas as pl
from jax.experimental.pallas import tpu as pltpu
from jax.experimental.pallas import tpu_sc as plsc

C0 = 2000
C1 = 20000
HEAD = 2002
HEAD_PAD = 2048
V0 = 18000
V1 = 80000
D1 = 256
D2 = 64
CHUNK = 4096
RB = 512

NC = 2
NS = 16
L = 16


def _gather_rows_body(t_hbm, w20_hbm, w21_hbm, out0_hbm, out1_hbm,
                      t_v, i0_v, i1_v, r0_v, r1_v, sem, *, bpw):
    wid = lax.axis_index("s") * NC + lax.axis_index("c")
    base = wid * bpw
    pltpu.sync_copy(t_hbm.at[pl.ds(base, bpw)], t_v)
    for j in range(bpw // L):
        t = t_v[pl.ds(j * L, L)]
        in0 = (t >= C0) & (t < C1)
        i0_v[pl.ds(j * L, L)] = jnp.where(in0, t - C0,
                                          jnp.bitwise_and(t, 8191))
        i1_v[pl.ds(j * L, L)] = jnp.where(t >= C1, t - C1, t) >> 1
    copies = []
    for k in range(bpw // 128):
        sl = pl.ds(k * 128, 128)
        copies.append(pltpu.async_copy(w20_hbm.at[i0_v.at[sl]],
                                       r0_v.at[sl], sem))
        copies.append(pltpu.async_copy(w21_hbm.at[i1_v.at[sl]],
                                       r1_v.at[sl], sem))
    for c in copies:
        c.wait()
    pltpu.sync_copy(r0_v, out0_hbm.at[pl.ds(base, bpw)])
    pltpu.sync_copy(r1_v, out1_hbm.at[pl.ds(base, bpw)])


def _gather_rows(t32, w20, w21):
    n = t32.shape[0]
    bpw = n // (NC * NS)
    body = functools.partial(_gather_rows_body, bpw=bpw)
    return pl.kernel(
        body,
        out_type=[jax.ShapeDtypeStruct((n, w20.shape[1]), jnp.float32),
                  jax.ShapeDtypeStruct((n, w21.shape[1]), jnp.float32)],
        mesh=plsc.VectorSubcoreMesh(core_axis_name="c", subcore_axis_name="s",
                                    num_cores=NC, num_subcores=NS),
        scratch_types=[pltpu.VMEM((bpw,), jnp.int32),
                       pltpu.VMEM((bpw,), jnp.int32),
                       pltpu.VMEM((bpw,), jnp.int32),
                       pltpu.VMEM((bpw, w20.shape[1]), jnp.float32),
                       pltpu.VMEM((bpw, w21.shape[1]), jnp.float32),
                       pltpu.SemaphoreType.DMA],
    )(t32, w20, w21)


def _main_kernel(x_ref, t_ref, wh_ref, w10_ref, w20_ref, w11_ref, w21_ref,
                 head_ref, lse0_ref, lse1_ref, h0_ref, h1_ref, *, rb):
    x = x_ref[...].astype(jnp.bfloat16)
    t = t_ref[0]

    h0 = jax.lax.dot_general(x, w10_ref[...], (((1,), (1,)), ((), ())),
                             preferred_element_type=jnp.float32)
    h0 = h0.astype(jnp.bfloat16)
    h1 = jax.lax.dot_general(x, w11_ref[...], (((1,), (1,)), ((), ())),
                             preferred_element_type=jnp.float32)
    h1 = h1.astype(jnp.bfloat16)
    h0_ref[...] = h0
    h1_ref[...] = h1

    lg = jax.lax.dot_general(x, wh_ref[...], (((1,), (1,)), ((), ())),
                             preferred_element_type=jnp.float32)
    col = jax.lax.broadcasted_iota(jnp.int32, (rb, HEAD_PAD), 1)
    s = jnp.sum(jnp.exp(lg), axis=1, keepdims=True)
    s = s - (HEAD_PAD - HEAD)
    lse_h = jnp.log(s)
    gidx = jnp.where(t < C0, t, jnp.where(t < C1, C0, C0 + 1))
    at_h = jnp.sum(jnp.where(col == gidx, lg, 0.0), axis=1, keepdims=True)
    head_ref[...] = at_h - lse_h

    def tail_lse(h, w_ref, vocab):
        nchunks = w_ref.shape[0] // CHUNK

        def body(c, s):
            w = w_ref[pl.ds(c * CHUNK, CHUNK), :]
            lg = jax.lax.dot_general(h, w, (((1,), (1,)), ((), ())),
                                     preferred_element_type=jnp.float32)
            return s + jnp.sum(jnp.exp(lg), axis=1, keepdims=True)

        zero = jnp.zeros((rb, 1), dtype=jnp.float32)
        s = jax.lax.fori_loop(0, nchunks, body, zero, unroll=2)
        s = s - (nchunks * CHUNK - vocab)
        return jnp.log(s)

    lse0_ref[...] = tail_lse(h0, w20_ref, V0)
    lse1_ref[...] = tail_lse(h1, w21_ref, V1)


def _combine_kernel(t_ref, head_ref, lse0_ref, lse1_ref, h0_ref, h1_ref,
                    w0r_ref, w1r_ref, out_ref):
    t = t_ref[0]

    def dot_rows(h, wr):
        wb = wr.astype(jnp.bfloat16).astype(jnp.float32)
        return jnp.sum(h.astype(jnp.float32) * wb, axis=1, keepdims=True)

    w1r = w1r_ref[...]
    odd = jnp.bitwise_and(t, 1) == 1
    w1sel = jnp.where(odd, w1r[:, D2:], w1r[:, :D2])
    local0 = dot_rows(h0_ref[...], w0r_ref[...]) - lse0_ref[...]
    local1 = dot_rows(h1_ref[...], w1sel) - lse1_ref[...]
    in0 = (t >= C0) & (t < C1)
    in1 = t >= C1
    out_ref[...] = head_ref[...] \
        + jnp.where(in0, local0, 0.0) \
        + jnp.where(in1, local1, 0.0)


def _pad_rows(w, n):
    return jnp.pad(w, ((0, n - w.shape[0]), (0, 0)))


@jax.jit
def kernel(input, target, W_head, W1_0, W2_0, W1_1, W2_1):
    n, d = input.shape
    rb = RB
    grid = n // rb

    t32 = target.astype(jnp.int32)
    x = input
    t3 = t32.reshape(grid, rb, 1)
    wh = _pad_rows(W_head, HEAD_PAD).astype(jnp.bfloat16)
    w10 = W1_0.astype(jnp.bfloat16)
    w20 = _pad_rows(W2_0, 5 * CHUNK).astype(jnp.bfloat16)
    w11 = W1_1.astype(jnp.bfloat16)
    w21 = _pad_rows(W2_1, 20 * CHUNK).astype(jnp.bfloat16)

    const = lambda shape: pl.BlockSpec(shape, lambda i: (0,) * len(shape))
    row = lambda width, dt: (pl.BlockSpec((rb, width), lambda i: (i, 0)),
                             jax.ShapeDtypeStruct((n, width), dt))
    out_specs, out_shapes = zip(
        row(1, jnp.float32),
        row(1, jnp.float32),
        row(1, jnp.float32),
        row(D1, jnp.bfloat16),
        row(D2, jnp.bfloat16),
    )
    head_t, lse0, lse1, h0, h1 = pl.pallas_call(
        functools.partial(_main_kernel, rb=rb),
        grid=(grid,),
        in_specs=[
            pl.BlockSpec((rb, d), lambda i: (i, 0)),
            pl.BlockSpec((1, rb, 1), lambda i: (i, 0, 0)),
            const(wh.shape),
            const(w10.shape),
            const(w20.shape),
            const(w11.shape),
            const(w21.shape),
        ],
        out_specs=list(out_specs),
        out_shape=list(out_shapes),
        compiler_params=pltpu.CompilerParams(
            dimension_semantics=("arbitrary",),
        ),
    )(x, t3, wh, w10, w20, w11, w21)

    w21v = W2_1.reshape(V1 // 2, 2 * D2)
    w0rows, w1rows = _gather_rows(t32, W2_0, w21v)

    cb = 2048
    cgrid = n // cb
    t3c = t32.reshape(cgrid, cb, 1)
    crow = lambda width: pl.BlockSpec((cb, width), lambda i: (i, 0))
    out = pl.pallas_call(
        _combine_kernel,
        grid=(cgrid,),
        in_specs=[
            pl.BlockSpec((1, cb, 1), lambda i: (i, 0, 0)),
            crow(1), crow(1), crow(1),
            crow(D1), crow(D2), crow(D1), crow(2 * D2),
        ],
        out_specs=crow(1),
        out_shape=jax.ShapeDtypeStruct((n, 1), jnp.float32),
        compiler_params=pltpu.CompilerParams(
            dimension_semantics=("arbitrary",),
        ),
    )(t3c, head_t, lse0, lse1, h0, h1, w0rows, w1rows)

    output = out.reshape(n)
    loss = (-output).mean()
    return output, loss


# speedup vs baseline: 1.1843x; 1.0577x over previous
import functools

import jax
import jax.numpy as jnp
from jax import lax
from jax.experimental import pallas as pl
from jax.experimental.pallas import tpu as pltpu
from jax.experimental.pallas import tpu_sc as plsc

C0 = 2000
C1 = 20000
HEAD = 2002
HEAD_PAD = 2048
V0 = 18000
V1 = 80000
D1 = 256
D2 = 64
CHUNK = 4096
RB = 512

NC = 2
NS = 16
L = 16


def _gather_rows_body(t_hbm, w20_hbm, w21_hbm, out0_hbm, out1_hbm,
                      t_v, i0_v, i1_v, r0_v, r1_v, sem, *, bpw):
    wid = lax.axis_index("s") * NC + lax.axis_index("c")
    base = wid * bpw
    pltpu.sync_copy(t_hbm.at[pl.ds(base, bpw)], t_v)
    for j in range(bpw // L):
        t = t_v[pl.ds(j * L, L)]
        in0 = (t >= C0) & (t < C1)
        i0_v[pl.ds(j * L, L)] = jnp.where(in0, t - C0,
                                          jnp.bitwise_and(t, 8191))
        i1_v[pl.ds(j * L, L)] = jnp.where(t >= C1, t - C1, t) >> 1
    copies = []
    for k in range(bpw // 128):
        sl = pl.ds(k * 128, 128)
        copies.append(pltpu.async_copy(w20_hbm.at[i0_v.at[sl]],
                                       r0_v.at[sl], sem))
        copies.append(pltpu.async_copy(w21_hbm.at[i1_v.at[sl]],
                                       r1_v.at[sl], sem))
    for c in copies:
        c.wait()
    pltpu.sync_copy(r0_v, out0_hbm.at[pl.ds(base, bpw)])
    pltpu.sync_copy(r1_v, out1_hbm.at[pl.ds(base, bpw)])


def _gather_rows(t32, w20, w21):
    n = t32.shape[0]
    bpw = n // (NC * NS)
    body = functools.partial(_gather_rows_body, bpw=bpw)
    return pl.kernel(
        body,
        out_type=[jax.ShapeDtypeStruct((n, w20.shape[1]), jnp.float32),
                  jax.ShapeDtypeStruct((n, w21.shape[1]), jnp.float32)],
        mesh=plsc.VectorSubcoreMesh(core_axis_name="c", subcore_axis_name="s",
                                    num_cores=NC, num_subcores=NS),
        scratch_types=[pltpu.VMEM((bpw,), jnp.int32),
                       pltpu.VMEM((bpw,), jnp.int32),
                       pltpu.VMEM((bpw,), jnp.int32),
                       pltpu.VMEM((bpw, w20.shape[1]), jnp.float32),
                       pltpu.VMEM((bpw, w21.shape[1]), jnp.float32),
                       pltpu.SemaphoreType.DMA],
    )(t32, w20, w21)


def _main_kernel(x_ref, t_ref, wh_ref, w10_ref, w20_ref, w11_ref, w21_ref,
                 head_ref, lse0_ref, lse1_ref, h0_ref, h1_ref, *, rb):
    x = x_ref[...].astype(jnp.bfloat16)
    t = t_ref[0]

    h0 = jax.lax.dot_general(x, w10_ref[...], (((1,), (1,)), ((), ())),
                             preferred_element_type=jnp.float32)
    h0 = h0.astype(jnp.bfloat16)
    h1 = jax.lax.dot_general(x, w11_ref[...], (((1,), (1,)), ((), ())),
                             preferred_element_type=jnp.float32)
    h1 = h1.astype(jnp.bfloat16)
    h0_ref[...] = h0
    h1_ref[...] = h1

    lg = jax.lax.dot_general(x, wh_ref[...], (((1,), (1,)), ((), ())),
                             preferred_element_type=jnp.float32)
    col = jax.lax.broadcasted_iota(jnp.int32, (rb, HEAD_PAD), 1)
    s = jnp.sum(jnp.exp(lg), axis=1, keepdims=True)
    s = s - (HEAD_PAD - HEAD)
    lse_h = jnp.log(s)
    gidx = jnp.where(t < C0, t, jnp.where(t < C1, C0, C0 + 1))
    at_h = jnp.sum(jnp.where(col == gidx, lg, 0.0), axis=1, keepdims=True)
    head_ref[...] = at_h - lse_h

    def tail_lse(h, w_ref, vocab):
        nchunks = w_ref.shape[0] // CHUNK

        def body(c, s):
            w = w_ref[pl.ds(c * CHUNK, CHUNK), :]
            lg = jax.lax.dot_general(h, w, (((1,), (1,)), ((), ())),
                                     preferred_element_type=jnp.float32)
            return s + jnp.sum(jnp.exp(lg), axis=1, keepdims=True)

        zero = jnp.zeros((rb, 1), dtype=jnp.float32)
        s = jax.lax.fori_loop(0, nchunks, body, zero, unroll=4)
        s = s - (nchunks * CHUNK - vocab)
        return jnp.log(s)

    lse0_ref[...] = tail_lse(h0, w20_ref, V0)
    lse1_ref[...] = tail_lse(h1, w21_ref, V1)


def _combine_kernel(t_ref, head_ref, lse0_ref, lse1_ref, h0_ref, h1_ref,
                    w0r_ref, w1r_ref, out_ref):
    t = t_ref[0]

    def dot_rows(h, wr):
        wb = wr.astype(jnp.bfloat16).astype(jnp.float32)
        return jnp.sum(h.astype(jnp.float32) * wb, axis=1, keepdims=True)

    w1r = w1r_ref[...]
    odd = jnp.bitwise_and(t, 1) == 1
    w1sel = jnp.where(odd, w1r[:, D2:], w1r[:, :D2])
    local0 = dot_rows(h0_ref[...], w0r_ref[...]) - lse0_ref[...]
    local1 = dot_rows(h1_ref[...], w1sel) - lse1_ref[...]
    in0 = (t >= C0) & (t < C1)
    in1 = t >= C1
    out_ref[...] = head_ref[...] \
        + jnp.where(in0, local0, 0.0) \
        + jnp.where(in1, local1, 0.0)


def _pad_rows(w, n):
    return jnp.pad(w, ((0, n - w.shape[0]), (0, 0)))


@jax.jit
def kernel(input, target, W_head, W1_0, W2_0, W1_1, W2_1):
    n, d = input.shape
    rb = RB
    grid = n // rb

    t32 = target.astype(jnp.int32)
    x = input
    t3 = t32.reshape(grid, rb, 1)
    wh = _pad_rows(W_head, HEAD_PAD).astype(jnp.bfloat16)
    w10 = W1_0.astype(jnp.bfloat16)
    w20 = _pad_rows(W2_0, 5 * CHUNK).astype(jnp.bfloat16)
    w11 = W1_1.astype(jnp.bfloat16)
    w21 = _pad_rows(W2_1, 20 * CHUNK).astype(jnp.bfloat16)

    const = lambda shape: pl.BlockSpec(shape, lambda i: (0,) * len(shape))
    row = lambda width, dt: (pl.BlockSpec((rb, width), lambda i: (i, 0)),
                             jax.ShapeDtypeStruct((n, width), dt))
    out_specs, out_shapes = zip(
        row(1, jnp.float32),
        row(1, jnp.float32),
        row(1, jnp.float32),
        row(D1, jnp.bfloat16),
        row(D2, jnp.bfloat16),
    )
    head_t, lse0, lse1, h0, h1 = pl.pallas_call(
        functools.partial(_main_kernel, rb=rb),
        grid=(grid,),
        in_specs=[
            pl.BlockSpec((rb, d), lambda i: (i, 0)),
            pl.BlockSpec((1, rb, 1), lambda i: (i, 0, 0)),
            const(wh.shape),
            const(w10.shape),
            const(w20.shape),
            const(w11.shape),
            const(w21.shape),
        ],
        out_specs=list(out_specs),
        out_shape=list(out_shapes),
        compiler_params=pltpu.CompilerParams(
            dimension_semantics=("arbitrary",),
        ),
    )(x, t3, wh, w10, w20, w11, w21)

    w21v = W2_1.reshape(V1 // 2, 2 * D2)
    w0rows, w1rows = _gather_rows(t32, W2_0, w21v)

    cb = 2048
    cgrid = n // cb
    t3c = t32.reshape(cgrid, cb, 1)
    crow = lambda width: pl.BlockSpec((cb, width), lambda i: (i, 0))
    out = pl.pallas_call(
        _combine_kernel,
        grid=(cgrid,),
        in_specs=[
            pl.BlockSpec((1, cb, 1), lambda i: (i, 0, 0)),
            crow(1), crow(1), crow(1),
            crow(D1), crow(D2), crow(D1), crow(2 * D2),
        ],
        out_specs=crow(1),
        out_shape=jax.ShapeDtypeStruct((n, 1), jnp.float32),
        compiler_params=pltpu.CompilerParams(
            dimension_semantics=("arbitrary",),
        ),
    )(t3c, head_t, lse0, lse1, h0, h1, w0rows, w1rows)

    output = out.reshape(n)
    loss = (-output).mean()
    return output, loss


# speedup vs baseline: 1.1919x; 1.0064x over previous
import functools

import jax
import jax.numpy as jnp
from jax import lax
from jax.experimental import pallas as pl
from jax.experimental.pallas import tpu as pltpu
from jax.experimental.pallas import tpu_sc as plsc

C0 = 2000
C1 = 20000
HEAD = 2002
HEAD_PAD = 2048
V0 = 18000
V1 = 80000
D1 = 256
D2 = 64
CHUNK = 4096
RB = 512

NC = 2
NS = 16
L = 16


def _gather_rows_body(t_hbm, w20_hbm, w21_hbm, out0_hbm, out1_hbm,
                      t_v, i0_v, i1_v, r0_v, r1_v, sem, *, bpw):
    wid = lax.axis_index("s") * NC + lax.axis_index("c")
    base = wid * bpw
    pltpu.sync_copy(t_hbm.at[pl.ds(base, bpw)], t_v)
    for j in range(bpw // L):
        t = t_v[pl.ds(j * L, L)]
        in0 = (t >= C0) & (t < C1)
        i0_v[pl.ds(j * L, L)] = jnp.where(in0, t - C0,
                                          jnp.bitwise_and(t, 8191))
        i1_v[pl.ds(j * L, L)] = jnp.where(t >= C1, t - C1, t) >> 1
    copies = []
    for k in range(bpw // 128):
        sl = pl.ds(k * 128, 128)
        copies.append(pltpu.async_copy(w20_hbm.at[i0_v.at[sl]],
                                       r0_v.at[sl], sem))
        copies.append(pltpu.async_copy(w21_hbm.at[i1_v.at[sl]],
                                       r1_v.at[sl], sem))
    for c in copies:
        c.wait()
    pltpu.sync_copy(r0_v, out0_hbm.at[pl.ds(base, bpw)])
    pltpu.sync_copy(r1_v, out1_hbm.at[pl.ds(base, bpw)])


def _gather_rows(t32, w20, w21):
    n = t32.shape[0]
    bpw = n // (NC * NS)
    body = functools.partial(_gather_rows_body, bpw=bpw)
    return pl.kernel(
        body,
        out_type=[jax.ShapeDtypeStruct((n, w20.shape[1]), jnp.float32),
                  jax.ShapeDtypeStruct((n, w21.shape[1]), jnp.float32)],
        mesh=plsc.VectorSubcoreMesh(core_axis_name="c", subcore_axis_name="s",
                                    num_cores=NC, num_subcores=NS),
        scratch_types=[pltpu.VMEM((bpw,), jnp.int32),
                       pltpu.VMEM((bpw,), jnp.int32),
                       pltpu.VMEM((bpw,), jnp.int32),
                       pltpu.VMEM((bpw, w20.shape[1]), jnp.float32),
                       pltpu.VMEM((bpw, w21.shape[1]), jnp.float32),
                       pltpu.SemaphoreType.DMA],
    )(t32, w20, w21)


def _main_kernel(x_ref, t_ref, wh_ref, w10_ref, w20_ref, w11_ref, w21_ref,
                 head_ref, lse0_ref, lse1_ref, h0_ref, h1_ref, *, rb):
    x = x_ref[...].astype(jnp.bfloat16)
    t = t_ref[0]

    h0 = jax.lax.dot_general(x, w10_ref[...], (((1,), (1,)), ((), ())),
                             preferred_element_type=jnp.float32)
    h0 = h0.astype(jnp.bfloat16)
    h1 = jax.lax.dot_general(x, w11_ref[...], (((1,), (1,)), ((), ())),
                             preferred_element_type=jnp.float32)
    h1 = h1.astype(jnp.bfloat16)
    h0_ref[...] = h0
    h1_ref[...] = h1

    lg = jax.lax.dot_general(x, wh_ref[...], (((1,), (1,)), ((), ())),
                             preferred_element_type=jnp.float32)
    col = jax.lax.broadcasted_iota(jnp.int32, (rb, HEAD_PAD), 1)
    s = jnp.sum(jnp.exp(lg), axis=1, keepdims=True)
    s = s - (HEAD_PAD - HEAD)
    lse_h = jnp.log(s)
    gidx = jnp.where(t < C0, t, jnp.where(t < C1, C0, C0 + 1))
    at_h = jnp.sum(jnp.where(col == gidx, lg, 0.0), axis=1, keepdims=True)
    head_ref[...] = at_h - lse_h

    def tail_lse(h, w_ref, vocab):
        nchunks = w_ref.shape[0] // CHUNK

        def body(c, s):
            w = w_ref[pl.ds(c * CHUNK, CHUNK), :]
            lg = jax.lax.dot_general(h, w, (((1,), (1,)), ((), ())),
                                     preferred_element_type=jnp.float32)
            return s + jnp.sum(jnp.exp(lg), axis=1, keepdims=True)

        zero = jnp.zeros((rb, 1), dtype=jnp.float32)
        s = jax.lax.fori_loop(0, nchunks, body, zero, unroll=5)
        s = s - (nchunks * CHUNK - vocab)
        return jnp.log(s)

    lse0_ref[...] = tail_lse(h0, w20_ref, V0)
    lse1_ref[...] = tail_lse(h1, w21_ref, V1)


def _combine_kernel(t_ref, head_ref, lse0_ref, lse1_ref, h0_ref, h1_ref,
                    w0r_ref, w1r_ref, out_ref):
    t = t_ref[0]

    def dot_rows(h, wr):
        wb = wr.astype(jnp.bfloat16).astype(jnp.float32)
        return jnp.sum(h.astype(jnp.float32) * wb, axis=1, keepdims=True)

    w1r = w1r_ref[...]
    odd = jnp.bitwise_and(t, 1) == 1
    w1sel = jnp.where(odd, w1r[:, D2:], w1r[:, :D2])
    local0 = dot_rows(h0_ref[...], w0r_ref[...]) - lse0_ref[...]
    local1 = dot_rows(h1_ref[...], w1sel) - lse1_ref[...]
    in0 = (t >= C0) & (t < C1)
    in1 = t >= C1
    out_ref[...] = head_ref[...] \
        + jnp.where(in0, local0, 0.0) \
        + jnp.where(in1, local1, 0.0)


def _pad_rows(w, n):
    return jnp.pad(w, ((0, n - w.shape[0]), (0, 0)))


@jax.jit
def kernel(input, target, W_head, W1_0, W2_0, W1_1, W2_1):
    n, d = input.shape
    rb = RB
    grid = n // rb

    t32 = target.astype(jnp.int32)
    x = input
    t3 = t32.reshape(grid, rb, 1)
    wh = _pad_rows(W_head, HEAD_PAD).astype(jnp.bfloat16)
    w10 = W1_0.astype(jnp.bfloat16)
    w20 = _pad_rows(W2_0, 5 * CHUNK).astype(jnp.bfloat16)
    w11 = W1_1.astype(jnp.bfloat16)
    w21 = _pad_rows(W2_1, 20 * CHUNK).astype(jnp.bfloat16)

    const = lambda shape: pl.BlockSpec(shape, lambda i: (0,) * len(shape))
    row = lambda width, dt: (pl.BlockSpec((rb, width), lambda i: (i, 0)),
                             jax.ShapeDtypeStruct((n, width), dt))
    out_specs, out_shapes = zip(
        row(1, jnp.float32),
        row(1, jnp.float32),
        row(1, jnp.float32),
        row(D1, jnp.bfloat16),
        row(D2, jnp.bfloat16),
    )
    head_t, lse0, lse1, h0, h1 = pl.pallas_call(
        functools.partial(_main_kernel, rb=rb),
        grid=(grid,),
        in_specs=[
            pl.BlockSpec((rb, d), lambda i: (i, 0)),
            pl.BlockSpec((1, rb, 1), lambda i: (i, 0, 0)),
            const(wh.shape),
            const(w10.shape),
            const(w20.shape),
            const(w11.shape),
            const(w21.shape),
        ],
        out_specs=list(out_specs),
        out_shape=list(out_shapes),
        compiler_params=pltpu.CompilerParams(
            dimension_semantics=("arbitrary",),
        ),
    )(x, t3, wh, w10, w20, w11, w21)

    w21v = W2_1.reshape(V1 // 2, 2 * D2)
    w0rows, w1rows = _gather_rows(t32, W2_0, w21v)

    cb = 2048
    cgrid = n // cb
    t3c = t32.reshape(cgrid, cb, 1)
    crow = lambda width: pl.BlockSpec((cb, width), lambda i: (i, 0))
    out = pl.pallas_call(
        _combine_kernel,
        grid=(cgrid,),
        in_specs=[
            pl.BlockSpec((1, cb, 1), lambda i: (i, 0, 0)),
            crow(1), crow(1), crow(1),
            crow(D1), crow(D2), crow(D1), crow(2 * D2),
        ],
        out_specs=crow(1),
        out_shape=jax.ShapeDtypeStruct((n, 1), jnp.float32),
        compiler_params=pltpu.CompilerParams(
            dimension_semantics=("arbitrary",),
        ),
    )(t3c, head_t, lse0, lse1, h0, h1, w0rows, w1rows)

    output = out.reshape(n)
    loss = (-output).mean()
    return output, loss
